# baseline (device time: 304876 ns/iter reference)
import jax
import jax.numpy as jnp
from jax import lax
from jax.experimental import pallas as pl
from jax.experimental.pallas import tpu as pltpu

N_DEV = 4


def kernel(x, w_mat, scale_x, scale_w):
    m_total, k_per = x.shape
    _, n = w_mat.shape
    m_per = m_total // N_DEV

    scale = (scale_x * scale_w).reshape(1, 1)

    def body(x_ref, w_ref, s_ref, out_ref, comm_ref, send_sems, recv_sems):
        my = lax.axis_index("i")
        left = lax.rem(my + N_DEV - 1, N_DEV)
        right = lax.rem(my + 1, N_DEV)

        barrier_sem = pltpu.get_barrier_semaphore()
        for nbr in (left, right):
            pl.semaphore_signal(
                barrier_sem, inc=1,
                device_id=(nbr,), device_id_type=pl.DeviceIdType.MESH,
            )
        pl.semaphore_wait(barrier_sem, 2)

        def partial(c):
            xc = x_ref[pl.ds(c * m_per, m_per), :]
            return lax.dot(xc, w_ref[...], preferred_element_type=jnp.int32)

        c0 = lax.rem(my + N_DEV - 1, N_DEV)
        comm_ref[0, :, :] = partial(c0)

        for h in range(N_DEV - 1):
            send_slot = h % 2
            recv_slot = (h + 1) % 2
            rdma = pltpu.make_async_remote_copy(
                src_ref=comm_ref.at[send_slot],
                dst_ref=comm_ref.at[recv_slot],
                send_sem=send_sems.at[send_slot],
                recv_sem=recv_sems.at[recv_slot],
                device_id=(right,),
                device_id_type=pl.DeviceIdType.MESH,
            )
            rdma.start()
            rdma.wait()

            c = lax.rem(my + 2 * N_DEV - 2 - h, N_DEV)
            if h < N_DEV - 2:
                comm_ref[recv_slot, :, :] = comm_ref[recv_slot, :, :] + partial(c)
            else:
                acc = comm_ref[recv_slot, :, :] + partial(c)
                out_ref[...] = jnp.maximum(
                    acc.astype(jnp.float32) * s_ref[0, 0], 0.0
                )

    return pl.pallas_call(
        body,
        out_shape=jax.ShapeDtypeStruct((m_per, n), jnp.float32),
        in_specs=[
            pl.BlockSpec(memory_space=pltpu.VMEM),
            pl.BlockSpec(memory_space=pltpu.VMEM),
            pl.BlockSpec(memory_space=pltpu.VMEM),
        ],
        out_specs=pl.BlockSpec(memory_space=pltpu.VMEM),
        scratch_shapes=[
            pltpu.VMEM((2, m_per, n), jnp.int32),
            pltpu.SemaphoreType.DMA((2,)),
            pltpu.SemaphoreType.DMA((2,)),
        ],
        compiler_params=pltpu.CompilerParams(collective_id=0),
    )(x, w_mat, scale)


# device time: 159079 ns/iter; 1.9165x vs baseline; 1.9165x over previous
import jax
import jax.numpy as jnp
from jax import lax
from jax.experimental import pallas as pl
from jax.experimental.pallas import tpu as pltpu

N_DEV = 4


def kernel(x, w_mat, scale_x, scale_w):
    m_total, k_per = x.shape
    _, n = w_mat.shape
    m_per = m_total // N_DEV
    n_half = n // 2

    scale = (scale_x * scale_w).reshape(1, 1)

    def body(x_ref, w_ref, s_ref, out_ref,
             comm_a, comm_b, send_sems_a, recv_sems_a,
             send_sems_b, recv_sems_b):
        my = lax.axis_index("i")
        left = lax.rem(my + N_DEV - 1, N_DEV)
        right = lax.rem(my + 1, N_DEV)

        barrier_sem = pltpu.get_barrier_semaphore()
        for nbr in (left, right):
            pl.semaphore_signal(
                barrier_sem, inc=1,
                device_id=(nbr,), device_id_type=pl.DeviceIdType.MESH,
            )
        pl.semaphore_wait(barrier_sem, 2)

        def partial_a(c):
            xc = x_ref[pl.ds(c * m_per, m_per), :]
            return lax.dot(xc, w_ref[:, :n_half],
                           preferred_element_type=jnp.int32)

        def partial_b(c):
            xc = x_ref[pl.ds(c * m_per, m_per), :]
            return lax.dot(xc, w_ref[:, n_half:],
                           preferred_element_type=jnp.int32)

        comm_a[0, :, :] = partial_a(lax.rem(my + N_DEV - 1, N_DEV))
        comm_b[0, :, :] = partial_b(lax.rem(my + 1, N_DEV))

        for h in range(N_DEV - 1):
            send_slot = h % 2
            recv_slot = (h + 1) % 2
            rdma_a = pltpu.make_async_remote_copy(
                src_ref=comm_a.at[send_slot],
                dst_ref=comm_a.at[recv_slot],
                send_sem=send_sems_a.at[send_slot],
                recv_sem=recv_sems_a.at[recv_slot],
                device_id=(right,),
                device_id_type=pl.DeviceIdType.MESH,
            )
            rdma_b = pltpu.make_async_remote_copy(
                src_ref=comm_b.at[send_slot],
                dst_ref=comm_b.at[recv_slot],
                send_sem=send_sems_b.at[send_slot],
                recv_sem=recv_sems_b.at[recv_slot],
                device_id=(left,),
                device_id_type=pl.DeviceIdType.MESH,
            )
            rdma_a.start()
            rdma_b.start()

            ca = lax.rem(my + 2 * N_DEV - 2 - h, N_DEV)
            cb = lax.rem(my + 2 + h, N_DEV)
            pa = partial_a(ca)
            pb = partial_b(cb)

            rdma_a.wait()
            rdma_b.wait()

            if h < N_DEV - 2:
                comm_a[recv_slot, :, :] = comm_a[recv_slot, :, :] + pa
                comm_b[recv_slot, :, :] = comm_b[recv_slot, :, :] + pb
            else:
                s = s_ref[0, 0]
                out_ref[:, :n_half] = jnp.maximum(
                    (comm_a[recv_slot, :, :] + pa).astype(jnp.float32) * s, 0.0
                )
                out_ref[:, n_half:] = jnp.maximum(
                    (comm_b[recv_slot, :, :] + pb).astype(jnp.float32) * s, 0.0
                )

    return pl.pallas_call(
        body,
        out_shape=jax.ShapeDtypeStruct((m_per, n), jnp.float32),
        in_specs=[
            pl.BlockSpec(memory_space=pltpu.VMEM),
            pl.BlockSpec(memory_space=pltpu.VMEM),
            pl.BlockSpec(memory_space=pltpu.VMEM),
        ],
        out_specs=pl.BlockSpec(memory_space=pltpu.VMEM),
        scratch_shapes=[
            pltpu.VMEM((2, m_per, n_half), jnp.int32),
            pltpu.VMEM((2, m_per, n_half), jnp.int32),
            pltpu.SemaphoreType.DMA((2,)),
            pltpu.SemaphoreType.DMA((2,)),
            pltpu.SemaphoreType.DMA((2,)),
            pltpu.SemaphoreType.DMA((2,)),
        ],
        compiler_params=pltpu.CompilerParams(collective_id=0),
    )(x, w_mat, scale)


# device time: 150760 ns/iter; 2.0223x vs baseline; 1.0552x over previous
import jax
import jax.numpy as jnp
from jax import lax
from jax.experimental import pallas as pl
from jax.experimental.pallas import tpu as pltpu

N_DEV = 4
HOPS = N_DEV - 1
SLOTS = 3
SUBS = 2


def kernel(x, w_mat, scale_x, scale_w):
    m_total, k_per = x.shape
    _, n = w_mat.shape
    m_per = m_total // N_DEV
    n_half = n // 2
    n_sub = n_half // SUBS

    scale = (scale_x * scale_w).reshape(1, 1)

    def body(x_ref, w_ref, s_ref, out_ref,
             comm_a, comm_b, ssem_a, rsem_a, ssem_b, rsem_b):
        my = lax.axis_index("i")
        left = lax.rem(my + N_DEV - 1, N_DEV)
        right = lax.rem(my + 1, N_DEV)

        barrier_sem = pltpu.get_barrier_semaphore()
        for nbr in (left, right):
            pl.semaphore_signal(
                barrier_sem, inc=1,
                device_id=(nbr,), device_id_type=pl.DeviceIdType.MESH,
            )
        pl.semaphore_wait(barrier_sem, 2)

        def pa(c, j):
            xc = x_ref[pl.ds(c * m_per, m_per), :]
            return lax.dot(xc, w_ref[:, j * n_sub:(j + 1) * n_sub],
                           preferred_element_type=jnp.int32)

        def pb(c, j):
            xc = x_ref[pl.ds(c * m_per, m_per), :]
            return lax.dot(
                xc, w_ref[:, n_half + j * n_sub:n_half + (j + 1) * n_sub],
                preferred_element_type=jnp.int32)

        def copy_a(h, j):
            return pltpu.make_async_remote_copy(
                src_ref=comm_a.at[h % SLOTS, j],
                dst_ref=comm_a.at[(h + 1) % SLOTS, j],
                send_sem=ssem_a.at[h, j],
                recv_sem=rsem_a.at[h, j],
                device_id=(right,),
                device_id_type=pl.DeviceIdType.MESH,
            )

        def copy_b(h, j):
            return pltpu.make_async_remote_copy(
                src_ref=comm_b.at[h % SLOTS, j],
                dst_ref=comm_b.at[(h + 1) % SLOTS, j],
                send_sem=ssem_b.at[h, j],
                recv_sem=rsem_b.at[h, j],
                device_id=(left,),
                device_id_type=pl.DeviceIdType.MESH,
            )

        def ca(h):
            return lax.rem(my + 2 * N_DEV - 1 - h, N_DEV)

        def cb(h):
            return lax.rem(my + 1 + h, N_DEV)

        sends = []

        for j in range(SUBS):
            comm_a[0, j] = pa(ca(0), j)
            r = copy_a(0, j)
            r.start()
            sends.append(r)
            comm_b[0, j] = pb(cb(0), j)
            r = copy_b(0, j)
            r.start()
            sends.append(r)

        for h in range(HOPS):
            nxt = (h + 1) % SLOTS
            last = h == HOPS - 1
            for j in range(SUBS):
                va = pa(ca(h + 1), j)
                copy_a(h, j).wait_recv()
                if not last:
                    comm_a[nxt, j] = comm_a[nxt, j] + va
                    r = copy_a(h + 1, j)
                    r.start()
                    sends.append(r)
                else:
                    out_ref[:, j * n_sub:(j + 1) * n_sub] = jnp.maximum(
                        (comm_a[nxt, j] + va).astype(jnp.float32)
                        * s_ref[0, 0], 0.0)

                vb = pb(cb(h + 1), j)
                copy_b(h, j).wait_recv()
                if not last:
                    comm_b[nxt, j] = comm_b[nxt, j] + vb
                    r = copy_b(h + 1, j)
                    r.start()
                    sends.append(r)
                else:
                    out_ref[:, n_half + j * n_sub:
                            n_half + (j + 1) * n_sub] = jnp.maximum(
                        (comm_b[nxt, j] + vb).astype(jnp.float32)
                        * s_ref[0, 0], 0.0)

        for r in sends:
            r.wait_send()

    return pl.pallas_call(
        body,
        out_shape=jax.ShapeDtypeStruct((m_per, n), jnp.float32),
        in_specs=[
            pl.BlockSpec(memory_space=pltpu.VMEM),
            pl.BlockSpec(memory_space=pltpu.VMEM),
            pl.BlockSpec(memory_space=pltpu.VMEM),
        ],
        out_specs=pl.BlockSpec(memory_space=pltpu.VMEM),
        scratch_shapes=[
            pltpu.VMEM((SLOTS, SUBS, m_per, n_sub), jnp.int32),
            pltpu.VMEM((SLOTS, SUBS, m_per, n_sub), jnp.int32),
            pltpu.SemaphoreType.DMA((HOPS, SUBS)),
            pltpu.SemaphoreType.DMA((HOPS, SUBS)),
            pltpu.SemaphoreType.DMA((HOPS, SUBS)),
            pltpu.SemaphoreType.DMA((HOPS, SUBS)),
        ],
        compiler_params=pltpu.CompilerParams(collective_id=0),
    )(x, w_mat, scale)
